# Initial kernel scaffold; baseline (speedup 1.0000x reference)
import jax
import jax.numpy as jnp
from jax.experimental import pallas as pl


def kernel(pred_logits, pred_boxes, target_sizes, pos_map):
    raise NotImplementedError("write your pallas kernel here")


# speedup vs baseline: 23.9331x; 23.9331x over previous
import functools

import jax
import jax.numpy as jnp
from jax import lax
from jax.experimental import pallas as pl
from jax.experimental.pallas import tpu as pltpu
from jax.experimental.pallas import tpu_sc as plsc

B, N, T, C = 8, 20000, 256, 91
CP = 128
NUM_SELECT = 300
M_ROWS = 320
BLK = 2000

NW = 4
ROWS_W = 5120
NV_A = ROWS_W // 16
RPW = M_ROWS // NW
NV_B = RPW * CP // 16
CAP = 384
EPW = NUM_SELECT // NW


def _prob_body(logits_ref, pm_ref, prob_ref, rowmax_ref):
    x = logits_ref[0]
    s = jax.nn.sigmoid(x)
    p = jax.lax.dot_general(
        s, pm_ref[...],
        dimension_numbers=(((1,), (1,)), ((), ())),
        preferred_element_type=jnp.float32)
    col = jax.lax.broadcasted_iota(jnp.int32, (BLK, CP), 1)
    p = jnp.where(col < C, p, jnp.float32(-1e30))
    prob_ref[0] = p
    rowmax_ref[0, 0, :] = jnp.max(p, axis=1)


def _compute_prob(pred_logits, pos_map):
    pm_pad = jnp.zeros((CP, T), jnp.float32).at[:C].set(pos_map)
    grid = (B, N // BLK)
    prob, rowmax = pl.pallas_call(
        _prob_body,
        grid=grid,
        in_specs=[
            pl.BlockSpec((1, BLK, T), lambda b, i: (b, i, 0)),
            pl.BlockSpec((CP, T), lambda b, i: (0, 0)),
        ],
        out_specs=[
            pl.BlockSpec((1, BLK, CP), lambda b, i: (b, i, 0)),
            pl.BlockSpec((1, 1, BLK), lambda b, i: (b * (N // BLK) + i, 0, 0)),
        ],
        out_shape=[
            jax.ShapeDtypeStruct((B, N, CP), jnp.float32),
            jax.ShapeDtypeStruct((B * (N // BLK), 1, BLK), jnp.float32),
        ],
    )(pred_logits, pm_pad)
    return prob, rowmax.reshape(B, N)


def _iota16():
    return lax.iota(jnp.int32, 16)


def _u32key(x):
    b = lax.bitcast_convert_type(x, jnp.int32)
    u = b ^ ((b >> 31) | jnp.int32(-2147483648))
    return lax.bitcast_convert_type(u, jnp.uint32)


def _splat_u32(s_i32):
    return lax.bitcast_convert_type(jnp.full((16,), s_i32, jnp.int32),
                                    jnp.uint32)


def _extract(vec, lane, fill):
    return jnp.max(jnp.where(_iota16() == lane, vec, fill))


def _sc_body(prob_hbm, rowmax_hbm, boxes_hbm, ts_hbm,
             scores_out, labels_out, boxes_out,
             rv, hist, comb, stg4, cnt_v, buf_i, buf_f,
             stg_cnt, stg_i4, stg_f4, cand_rows, idx80, cprob,
             vals_s, flats_s, res_pos, res_score, res_label, res_row,
             stg_pos, stg_sc, stg_lab, stg_row,
             srt_score, srt_label, srt_row, bxall, bxo, ts_s, sem,
             sh_hist, sh_cnt, sh_buf_i, sh_buf_f,
             sh_pos, sh_sc, sh_lab, sh_row):
    c = lax.axis_index("c")
    s = lax.axis_index("s")
    b = c * 4 + s // 4
    lb = s // 4
    j = s % 4
    iota = _iota16()
    ones = jnp.ones((16,), jnp.int32)
    zeros16 = jnp.zeros((16,), jnp.int32)

    pltpu.sync_copy(rowmax_hbm.at[pl.ds((b * NW + j) * ROWS_W, ROWS_W)], rv)

    def load_a(i):
        return _u32key(plsc.load_gather(rv, [i * 16 + iota]))

    def ids_a(i):
        return j * ROWS_W + i * 16 + iota

    def load_b(i):
        q = i * 16 + iota
        x = plsc.load_gather(cprob, [q // CP, q % CP])
        return _u32key(x)

    def ids_b(i):
        return j * (RPW * CP) + i * 16 + iota

    def stage_count(off):
        cnt_v[pl.ds(0, 16)] = jnp.where(iota == 0, off, 0)
        pltpu.sync_copy(cnt_v, sh_cnt.at[lb, j])

    def read_counts():
        pltpu.sync_copy(sh_cnt.at[lb], stg_cnt)
        def cnt(jj):
            v = plsc.load_gather(stg_cnt,
                                 [jnp.full((16,), jj, jnp.int32), iota])
            return v[0]
        c0, c1, c2 = cnt(0), cnt(1), cnt(2)
        return c0, c0 + c1, c0 + c1 + c2

    def select(nv, loader, idfun, m_target):
        pfx = jnp.int32(0)
        r = jnp.int32(m_target)
        for level in range(4):
            shift = 24 - 8 * level

            def clr(i, _):
                plsc.store_scatter(hist, [i * 16 + iota], zeros16)
                return 0
            lax.fori_loop(0, 256, clr, 0)
            pfxs = _splat_u32(pfx)

            def hbody(i, _):
                k = loader(i)
                bn = lax.convert_element_type(
                    (k >> jnp.uint32(shift)) & jnp.uint32(0xFF), jnp.int32)
                addr = iota * 256 + bn
                if level == 0:
                    plsc.addupdate_scatter(hist, [addr], ones)
                else:
                    m = (k >> jnp.uint32(shift + 8)) == pfxs
                    plsc.addupdate_scatter(hist, [addr], ones, mask=m)
                return 0
            lax.fori_loop(0, nv, hbody, 0)

            def lane_comb(i, _):
                acc = plsc.load_gather(hist, [i * 16 + iota])
                for l in range(1, 16):
                    acc = acc + plsc.load_gather(hist,
                                                 [l * 256 + i * 16 + iota])
                plsc.store_scatter(comb, [i * 16 + iota], acc)
                return 0
            lax.fori_loop(0, 16, lane_comb, 0)
            pltpu.sync_copy(comb, sh_hist.at[lb, j])
            plsc.subcore_barrier()
            pltpu.sync_copy(sh_hist.at[lb], stg4)

            def sum4(i, _):
                g = (plsc.load_gather(stg4, [jnp.full((16,), 0, jnp.int32),
                                             i * 16 + iota])
                     + plsc.load_gather(stg4, [jnp.full((16,), 1, jnp.int32),
                                               i * 16 + iota])
                     + plsc.load_gather(stg4, [jnp.full((16,), 2, jnp.int32),
                                               i * 16 + iota])
                     + plsc.load_gather(stg4, [jnp.full((16,), 3, jnp.int32),
                                               i * 16 + iota]))
                plsc.store_scatter(comb, [i * 16 + iota], g)
                return 0
            lax.fori_loop(0, 16, sum4, 0)
            plsc.subcore_barrier()

            def scan_body(t, carry):
                found, run, bn, gab = carry
                i = 15 - t
                h = plsc.load_gather(comb, [i * 16 + iota])
                rh = lax.rev(h, (0,))
                sc = plsc.cumsum(rh)
                tot = jnp.max(sc)
                cross = (run + sc) >= r
                anyc = jnp.max(jnp.where(cross, 1, 0))
                hit = (anyc > 0) & (found == 0)
                lstar = jnp.min(jnp.where(cross, iota, 16))
                sstar = _extract(sc, lstar, 0)
                rstar = _extract(rh, lstar, 0)
                bn = jnp.where(hit, i * 16 + 15 - lstar, bn)
                gab = jnp.where(hit, run + sstar - rstar, gab)
                found = jnp.maximum(found, anyc)
                run = jnp.where(found > 0, run, run + tot)
                return (found, run, bn, gab)
            _, _, bn, gab = lax.fori_loop(
                0, 16, scan_body,
                (jnp.int32(0), jnp.int32(0), jnp.int32(0), jnp.int32(0)))
            pfx = (pfx << 8) | bn
            r = r - gab

        vstar = _splat_u32(pfx)

        def tie_body(i, off):
            k = loader(i)
            m = k == vstar
            ps = plsc.cumsum(lax.convert_element_type(m, jnp.int32))
            idx = off + ps - 1
            ms = m & (idx < CAP)
            plsc.store_scatter(buf_i, [idx], idfun(i), mask=ms)
            return off + jnp.max(ps)
        off = lax.fori_loop(0, nv, tie_body, jnp.int32(0))
        stage_count(off)
        pltpu.sync_copy(buf_i, sh_buf_i.at[lb, j])
        plsc.subcore_barrier()
        p1, p2, p3 = read_counts()
        pltpu.sync_copy(sh_buf_i.at[lb], stg_i4)
        own = (lax.convert_element_type(r > p1, jnp.int32)
               + lax.convert_element_type(r > p2, jnp.int32)
               + lax.convert_element_type(r > p3, jnp.int32))
        pre = jnp.where(own == 0, 0,
                        jnp.where(own == 1, p1, jnp.where(own == 2, p2, p3)))
        idstar_v = plsc.load_gather(
            stg_i4, [jnp.full((16,), own, jnp.int32),
                     jnp.full((16,), r - 1 - pre, jnp.int32)])
        plsc.subcore_barrier()
        return vstar, idstar_v

    vstar_a, rstar_a = select(NV_A, load_a, ids_a, M_ROWS)

    def fa_body(i, off):
        k = load_a(i)
        ids = ids_a(i)
        m = (k > vstar_a) | ((k == vstar_a) & (ids <= rstar_a))
        ps = plsc.cumsum(lax.convert_element_type(m, jnp.int32))
        idx = off + ps - 1
        plsc.store_scatter(buf_i, [idx], ids, mask=m)
        return off + jnp.max(ps)
    off = lax.fori_loop(0, NV_A, fa_body, jnp.int32(0))
    stage_count(off)
    pltpu.sync_copy(buf_i, sh_buf_i.at[lb, j])
    plsc.subcore_barrier()
    p1, p2, p3 = read_counts()
    pltpu.sync_copy(sh_buf_i.at[lb], stg_i4)

    def build_cand(i, _):
        p = i * 16 + iota
        own = (lax.convert_element_type(p >= p1, jnp.int32)
               + lax.convert_element_type(p >= p2, jnp.int32)
               + lax.convert_element_type(p >= p3, jnp.int32))
        pre = jnp.where(own == 0, 0,
                        jnp.where(own == 1, p1, jnp.where(own == 2, p2, p3)))
        rows = plsc.load_gather(stg_i4, [own, p - pre])
        plsc.store_scatter(cand_rows, [p], rows)
        return 0
    lax.fori_loop(0, M_ROWS // 16, build_cand, 0)
    plsc.subcore_barrier()

    for i in range(RPW // 16):
        rows = plsc.load_gather(cand_rows, [j * RPW + i * 16 + iota])
        idx80[pl.ds(i * 16, 16)] = rows + b * N
    pltpu.async_copy(prob_hbm.at[idx80], cprob, sem).wait()

    vstar_b, fstar_b = select(NV_B, load_b, ids_b, NUM_SELECT)

    def fb_body(i, off):
        q = i * 16 + iota
        x = plsc.load_gather(cprob, [q // CP, q % CP])
        k = _u32key(x)
        f = ids_b(i)
        m = (k > vstar_b) | ((k == vstar_b) & (f <= fstar_b))
        ps = plsc.cumsum(lax.convert_element_type(m, jnp.int32))
        idx = off + ps - 1
        plsc.store_scatter(buf_i, [idx], f, mask=m)
        plsc.store_scatter(buf_f, [idx], x, mask=m)
        return off + jnp.max(ps)
    off = lax.fori_loop(0, NV_B, fb_body, jnp.int32(0))
    stage_count(off)
    pltpu.sync_copy(buf_i, sh_buf_i.at[lb, j])
    pltpu.sync_copy(buf_f, sh_buf_f.at[lb, j])
    plsc.subcore_barrier()
    p1, p2, p3 = read_counts()
    pltpu.sync_copy(sh_buf_i.at[lb], stg_i4)
    pltpu.sync_copy(sh_buf_f.at[lb], stg_f4)

    def build_sel(i, _):
        p = i * 16 + iota
        pc = jnp.minimum(p, NUM_SELECT - 1)
        own = (lax.convert_element_type(pc >= p1, jnp.int32)
               + lax.convert_element_type(pc >= p2, jnp.int32)
               + lax.convert_element_type(pc >= p3, jnp.int32))
        pre = jnp.where(own == 0, 0,
                        jnp.where(own == 1, p1, jnp.where(own == 2, p2, p3)))
        plsc.store_scatter(flats_s, [p],
                           plsc.load_gather(stg_i4, [own, pc - pre]))
        plsc.store_scatter(vals_s, [p],
                           plsc.load_gather(stg_f4, [own, pc - pre]))
        return 0
    lax.fori_loop(0, 304 // 16, build_sel, 0)
    plsc.subcore_barrier()

    myv = []
    myf = []
    for v in range(5):
        e = jnp.minimum(j * EPW + v * 16 + iota, NUM_SELECT - 1)
        myv.append(plsc.load_gather(vals_s, [e]))
        myf.append(plsc.load_gather(flats_s, [e]))

    def rank_body(t, cnts):
        ts16 = jnp.full((16,), t, jnp.int32)
        vs = plsc.load_gather(vals_s, [ts16])
        fs = plsc.load_gather(flats_s, [ts16])
        out = []
        for v in range(5):
            win = (vs > myv[v]) | ((vs == myv[v]) & (fs < myf[v]))
            out.append(cnts[v] + lax.convert_element_type(win, jnp.int32))
        return tuple(out)
    cnts = lax.fori_loop(0, NUM_SELECT, rank_body,
                         tuple(zeros16 for _ in range(5)))

    for v in range(5):
        f = myf[v]
        p = jnp.clip(f // CP, 0, M_ROWS - 1)
        row = plsc.load_gather(cand_rows, [p])
        res_pos[pl.ds(v * 16, 16)] = cnts[v]
        res_score[pl.ds(v * 16, 16)] = myv[v]
        res_label[pl.ds(v * 16, 16)] = f % CP
        res_row[pl.ds(v * 16, 16)] = row
    pltpu.sync_copy(res_pos, sh_pos.at[lb, j])
    pltpu.sync_copy(res_score, sh_sc.at[lb, j])
    pltpu.sync_copy(res_label, sh_lab.at[lb, j])
    pltpu.sync_copy(res_row, sh_row.at[lb, j])
    plsc.subcore_barrier()

    @pl.when(j == 0)
    def _():
        pltpu.sync_copy(ts_hbm, ts_s)
        pltpu.sync_copy(boxes_hbm.at[pl.ds(b * (N * 4), N * 4)], bxall)
        pltpu.sync_copy(sh_pos.at[lb], stg_pos)
        pltpu.sync_copy(sh_sc.at[lb], stg_sc)
        pltpu.sync_copy(sh_lab.at[lb], stg_lab)
        pltpu.sync_copy(sh_row.at[lb], stg_row)

        def zero_out(i, _):
            plsc.store_scatter(srt_score, [i * 16 + iota],
                               jnp.zeros((16,), jnp.float32))
            plsc.store_scatter(srt_label, [i * 16 + iota], zeros16)
            plsc.store_scatter(srt_row, [jnp.minimum(i * 16 + iota, 319)],
                               zeros16)
            return 0
        lax.fori_loop(0, 384 // 16, zero_out, 0)

        def scat(i, _):
            e = i * 16 + iota
            w = e // RPW
            l = e % RPW
            valid = l < EPW
            pos = plsc.load_gather(stg_pos, [w, l])
            sc = plsc.load_gather(stg_sc, [w, l])
            lab = plsc.load_gather(stg_lab, [w, l])
            rw = plsc.load_gather(stg_row, [w, l])
            posc = jnp.clip(pos, 0, NUM_SELECT - 1)
            plsc.store_scatter(srt_score, [posc], sc, mask=valid)
            plsc.store_scatter(srt_label, [posc], lab, mask=valid)
            plsc.store_scatter(srt_row, [posc], rw, mask=valid)
            return 0
        lax.fori_loop(0, M_ROWS // 16, scat, 0)

        tsv = ts_s[pl.ds(0, 16)]
        hh = _extract(tsv, b * 2, jnp.int32(-2147483648))
        ww = _extract(tsv, b * 2 + 1, jnp.int32(-2147483648))
        wv = jnp.full((16,), lax.convert_element_type(ww, jnp.float32))
        hv = jnp.full((16,), lax.convert_element_type(hh, jnp.float32))

        def box_body(i, _):
            q = i * 16 + iota
            orow = q // 4
            col = q % 4
            row = plsc.load_gather(srt_row, [orow])
            v1 = plsc.load_gather(bxall, [row * 4 + col])
            v2 = plsc.load_gather(bxall, [row * 4 + (col ^ 2)])
            low = col < 2
            xy = jnp.where(low, v1 - 0.5 * v2, v2 + 0.5 * v1)
            scale = jnp.where((col & 1) == 0, wv, hv)
            plsc.store_scatter(bxo, [q], xy * scale)
            return 0
        lax.fori_loop(0, 1280 // 16, box_body, 0)

        pltpu.sync_copy(srt_score, scores_out.at[pl.ds(b * 384, 384)])
        pltpu.sync_copy(srt_label, labels_out.at[pl.ds(b * 384, 384)])
        pltpu.sync_copy(bxo, boxes_out.at[pl.ds(b * 1280, 1280)])


def _sc_select(prob_flat, rowmax, pred_boxes, target_sizes):
    mesh = plsc.VectorSubcoreMesh(core_axis_name="c", subcore_axis_name="s")
    f32 = jnp.float32
    i32 = jnp.int32
    kern = pl.kernel(
        _sc_body,
        mesh=mesh,
        compiler_params=pltpu.CompilerParams(needs_layout_passes=False),
        out_type=[
            jax.ShapeDtypeStruct((B * 384,), f32),
            jax.ShapeDtypeStruct((B * 384,), i32),
            jax.ShapeDtypeStruct((B * 1280,), f32),
        ],
        scratch_types=[
            pltpu.VMEM((NV_A * 16,), f32),
            pltpu.VMEM((4096,), i32),
            pltpu.VMEM((256,), i32),
            pltpu.VMEM((4, 256), i32),
            pltpu.VMEM((128,), i32),
            pltpu.VMEM((CAP,), i32),
            pltpu.VMEM((CAP,), f32),
            pltpu.VMEM((4, 128), i32),
            pltpu.VMEM((4, CAP), i32),
            pltpu.VMEM((4, CAP), f32),
            pltpu.VMEM((M_ROWS,), i32),
            pltpu.VMEM((RPW,), i32),
            pltpu.VMEM((RPW, CP), f32),
            pltpu.VMEM((304,), f32),
            pltpu.VMEM((304,), i32),
            pltpu.VMEM((128,), i32),
            pltpu.VMEM((128,), f32),
            pltpu.VMEM((128,), i32),
            pltpu.VMEM((128,), i32),
            pltpu.VMEM((4, 128), i32),
            pltpu.VMEM((4, 128), f32),
            pltpu.VMEM((4, 128), i32),
            pltpu.VMEM((4, 128), i32),
            pltpu.VMEM((384,), f32),
            pltpu.VMEM((384,), i32),
            pltpu.VMEM((320,), i32),
            pltpu.VMEM((N * 4,), f32),
            pltpu.VMEM((1280,), f32),
            pltpu.VMEM((B * 2,), i32),
            pltpu.SemaphoreType.DMA,
            pltpu.VMEM_SHARED((4, 4, 256), i32),
            pltpu.VMEM_SHARED((4, 4, 128), i32),
            pltpu.VMEM_SHARED((4, 4, CAP), i32),
            pltpu.VMEM_SHARED((4, 4, CAP), f32),
            pltpu.VMEM_SHARED((4, 4, 128), i32),
            pltpu.VMEM_SHARED((4, 4, 128), f32),
            pltpu.VMEM_SHARED((4, 4, 128), i32),
            pltpu.VMEM_SHARED((4, 4, 128), i32),
        ],
    )
    return kern(prob_flat, rowmax, pred_boxes, target_sizes)


@jax.jit
def kernel(pred_logits, pred_boxes, target_sizes, pos_map):
    prob, rowmax = _compute_prob(pred_logits, pos_map)
    rowmax_pad = jnp.concatenate(
        [rowmax, jnp.full((B, NW * ROWS_W - N), -jnp.inf, jnp.float32)],
        axis=1).reshape(-1)
    scores_p, labels_p, boxes_p = _sc_select(
        prob.reshape(B * N, CP), rowmax_pad, pred_boxes.reshape(-1),
        target_sizes.reshape(-1))
    return (scores_p.reshape(B, 384)[:, :NUM_SELECT],
            labels_p.reshape(B, 384)[:, :NUM_SELECT],
            boxes_p.reshape(B, 320, 4)[:, :NUM_SELECT, :])


# speedup vs baseline: 32.0269x; 1.3382x over previous
import functools

import jax
import jax.numpy as jnp
from jax import lax
from jax.experimental import pallas as pl
from jax.experimental.pallas import tpu as pltpu
from jax.experimental.pallas import tpu_sc as plsc

B, N, T, C = 8, 20000, 256, 91
CP = 128
NUM_SELECT = 300
M_ROWS = 320
BLK = 10000

NW = 4
ROWS_W = 5120
NV_A = ROWS_W // 16
RPW = M_ROWS // NW
NV_B = RPW * CP // 16
CAP = 384
EPW = NUM_SELECT // NW


def _prob_body(logits_ref, pm_ref, prob_ref, rowmax_ref):
    x = logits_ref[0]
    s = jax.nn.sigmoid(x)
    p = jax.lax.dot_general(
        s, pm_ref[...],
        dimension_numbers=(((1,), (1,)), ((), ())),
        preferred_element_type=jnp.float32)
    col = jax.lax.broadcasted_iota(jnp.int32, (BLK, CP), 1)
    p = jnp.where(col < C, p, jnp.float32(-1e30))
    prob_ref[0] = p
    rowmax_ref[0, 0, :] = jnp.max(p.T, axis=0)


def _compute_prob(pred_logits, pos_map):
    pm_pad = jnp.zeros((CP, T), jnp.float32).at[:C].set(pos_map)
    grid = (B, N // BLK)
    prob, rowmax = pl.pallas_call(
        _prob_body,
        grid=grid,
        in_specs=[
            pl.BlockSpec((1, BLK, T), lambda b, i: (b, i, 0)),
            pl.BlockSpec((CP, T), lambda b, i: (0, 0)),
        ],
        out_specs=[
            pl.BlockSpec((1, BLK, CP), lambda b, i: (b, i, 0)),
            pl.BlockSpec((1, 1, BLK), lambda b, i: (b * (N // BLK) + i, 0, 0)),
        ],
        out_shape=[
            jax.ShapeDtypeStruct((B, N, CP), jnp.float32),
            jax.ShapeDtypeStruct((B * (N // BLK), 1, BLK), jnp.float32),
        ],
    )(pred_logits, pm_pad)
    return prob, rowmax.reshape(B, N)


def _iota16():
    return lax.iota(jnp.int32, 16)


def _u32key(x):
    b = lax.bitcast_convert_type(x, jnp.int32)
    u = b ^ ((b >> 31) | jnp.int32(-2147483648))
    return lax.bitcast_convert_type(u, jnp.uint32)


def _splat_u32(s_i32):
    return lax.bitcast_convert_type(jnp.full((16,), s_i32, jnp.int32),
                                    jnp.uint32)


def _extract(vec, lane, fill):
    return jnp.max(jnp.where(_iota16() == lane, vec, fill))


def _sc_body(prob_hbm, rowmax_hbm, boxes_hbm, ts_hbm,
             scores_out, labels_out, boxes_out,
             rv, hist, comb, stg4, cnt_v, buf_i, buf_f,
             stg_cnt, stg_i4, stg_f4, cand_rows, idx80, cprob,
             vals_s, flats_s, res_pos, res_score, res_label, res_row,
             stg_pos, stg_sc, stg_lab, stg_row,
             srt_score, srt_label, srt_row, bxall, bxo, ts_s, sem,
             sh_hist, sh_cnt, sh_buf_i, sh_buf_f,
             sh_pos, sh_sc, sh_lab, sh_row):
    c = lax.axis_index("c")
    s = lax.axis_index("s")
    b = c * 4 + s // 4
    lb = s // 4
    j = s % 4
    iota = _iota16()
    ones = jnp.ones((16,), jnp.int32)
    zeros16 = jnp.zeros((16,), jnp.int32)

    pltpu.sync_copy(rowmax_hbm.at[pl.ds((b * NW + j) * ROWS_W, ROWS_W)], rv)

    def load_a(i):
        return _u32key(plsc.load_gather(rv, [i * 16 + iota]))

    def ids_a(i):
        return j * ROWS_W + i * 16 + iota

    def load_b(i):
        q = i * 16 + iota
        x = plsc.load_gather(cprob, [q // CP, q % CP])
        return _u32key(x)

    def ids_b(i):
        return j * (RPW * CP) + i * 16 + iota

    def stage_count(off):
        cnt_v[pl.ds(0, 16)] = jnp.where(iota == 0, off, 0)
        pltpu.sync_copy(cnt_v, sh_cnt.at[lb, j])

    def read_counts():
        pltpu.sync_copy(sh_cnt.at[lb], stg_cnt)
        def cnt(jj):
            v = plsc.load_gather(stg_cnt,
                                 [jnp.full((16,), jj, jnp.int32), iota])
            return v[0]
        c0, c1, c2 = cnt(0), cnt(1), cnt(2)
        return c0, c0 + c1, c0 + c1 + c2

    def select(nv, loader, idfun, m_target):
        pfx = jnp.int32(0)
        r = jnp.int32(m_target)
        for level in range(4):
            shift = 24 - 8 * level

            def clr(i, _):
                plsc.store_scatter(hist, [i * 16 + iota], zeros16)
                return 0
            lax.fori_loop(0, 256, clr, 0)
            pfxs = _splat_u32(pfx)

            def hbody(i, _):
                k = loader(i)
                bn = lax.convert_element_type(
                    (k >> jnp.uint32(shift)) & jnp.uint32(0xFF), jnp.int32)
                addr = iota * 256 + bn
                if level == 0:
                    plsc.addupdate_scatter(hist, [addr], ones)
                else:
                    m = (k >> jnp.uint32(shift + 8)) == pfxs
                    plsc.addupdate_scatter(hist, [addr], ones, mask=m)
                return 0
            lax.fori_loop(0, nv, hbody, 0)

            def lane_comb(i, _):
                acc = plsc.load_gather(hist, [i * 16 + iota])
                for l in range(1, 16):
                    acc = acc + plsc.load_gather(hist,
                                                 [l * 256 + i * 16 + iota])
                plsc.store_scatter(comb, [i * 16 + iota], acc)
                return 0
            lax.fori_loop(0, 16, lane_comb, 0)
            pltpu.sync_copy(comb, sh_hist.at[lb, j])
            plsc.subcore_barrier()
            pltpu.sync_copy(sh_hist.at[lb], stg4)

            def sum4(i, _):
                g = (plsc.load_gather(stg4, [jnp.full((16,), 0, jnp.int32),
                                             i * 16 + iota])
                     + plsc.load_gather(stg4, [jnp.full((16,), 1, jnp.int32),
                                               i * 16 + iota])
                     + plsc.load_gather(stg4, [jnp.full((16,), 2, jnp.int32),
                                               i * 16 + iota])
                     + plsc.load_gather(stg4, [jnp.full((16,), 3, jnp.int32),
                                               i * 16 + iota]))
                plsc.store_scatter(comb, [i * 16 + iota], g)
                return 0
            lax.fori_loop(0, 16, sum4, 0)
            plsc.subcore_barrier()

            def scan_body(t, carry):
                found, run, bn, gab = carry
                i = 15 - t
                h = plsc.load_gather(comb, [i * 16 + iota])
                rh = lax.rev(h, (0,))
                sc = plsc.cumsum(rh)
                tot = jnp.max(sc)
                cross = (run + sc) >= r
                anyc = jnp.max(jnp.where(cross, 1, 0))
                hit = (anyc > 0) & (found == 0)
                lstar = jnp.min(jnp.where(cross, iota, 16))
                sstar = _extract(sc, lstar, 0)
                rstar = _extract(rh, lstar, 0)
                bn = jnp.where(hit, i * 16 + 15 - lstar, bn)
                gab = jnp.where(hit, run + sstar - rstar, gab)
                found = jnp.maximum(found, anyc)
                run = jnp.where(found > 0, run, run + tot)
                return (found, run, bn, gab)
            _, _, bn, gab = lax.fori_loop(
                0, 16, scan_body,
                (jnp.int32(0), jnp.int32(0), jnp.int32(0), jnp.int32(0)))
            pfx = (pfx << 8) | bn
            r = r - gab

        vstar = _splat_u32(pfx)

        def tie_body(i, off):
            k = loader(i)
            m = k == vstar
            ps = plsc.cumsum(lax.convert_element_type(m, jnp.int32))
            idx = off + ps - 1
            ms = m & (idx < CAP)
            plsc.store_scatter(buf_i, [idx], idfun(i), mask=ms)
            return off + jnp.max(ps)
        off = lax.fori_loop(0, nv, tie_body, jnp.int32(0))
        stage_count(off)
        pltpu.sync_copy(buf_i, sh_buf_i.at[lb, j])
        plsc.subcore_barrier()
        p1, p2, p3 = read_counts()
        pltpu.sync_copy(sh_buf_i.at[lb], stg_i4)
        own = (lax.convert_element_type(r > p1, jnp.int32)
               + lax.convert_element_type(r > p2, jnp.int32)
               + lax.convert_element_type(r > p3, jnp.int32))
        pre = jnp.where(own == 0, 0,
                        jnp.where(own == 1, p1, jnp.where(own == 2, p2, p3)))
        idstar_v = plsc.load_gather(
            stg_i4, [jnp.full((16,), own, jnp.int32),
                     jnp.full((16,), r - 1 - pre, jnp.int32)])
        plsc.subcore_barrier()
        return vstar, idstar_v

    vstar_a, rstar_a = select(NV_A, load_a, ids_a, M_ROWS)

    def fa_body(i, off):
        k = load_a(i)
        ids = ids_a(i)
        m = (k > vstar_a) | ((k == vstar_a) & (ids <= rstar_a))
        ps = plsc.cumsum(lax.convert_element_type(m, jnp.int32))
        idx = off + ps - 1
        plsc.store_scatter(buf_i, [idx], ids, mask=m)
        return off + jnp.max(ps)
    off = lax.fori_loop(0, NV_A, fa_body, jnp.int32(0))
    stage_count(off)
    pltpu.sync_copy(buf_i, sh_buf_i.at[lb, j])
    plsc.subcore_barrier()
    p1, p2, p3 = read_counts()
    pltpu.sync_copy(sh_buf_i.at[lb], stg_i4)

    def build_cand(i, _):
        p = i * 16 + iota
        own = (lax.convert_element_type(p >= p1, jnp.int32)
               + lax.convert_element_type(p >= p2, jnp.int32)
               + lax.convert_element_type(p >= p3, jnp.int32))
        pre = jnp.where(own == 0, 0,
                        jnp.where(own == 1, p1, jnp.where(own == 2, p2, p3)))
        rows = plsc.load_gather(stg_i4, [own, p - pre])
        plsc.store_scatter(cand_rows, [p], rows)
        return 0
    lax.fori_loop(0, M_ROWS // 16, build_cand, 0)
    plsc.subcore_barrier()

    for i in range(RPW // 16):
        rows = plsc.load_gather(cand_rows, [j * RPW + i * 16 + iota])
        idx80[pl.ds(i * 16, 16)] = rows + b * N
    pltpu.async_copy(prob_hbm.at[idx80], cprob, sem).wait()

    vstar_b, fstar_b = select(NV_B, load_b, ids_b, NUM_SELECT)

    def fb_body(i, off):
        q = i * 16 + iota
        x = plsc.load_gather(cprob, [q // CP, q % CP])
        k = _u32key(x)
        f = ids_b(i)
        m = (k > vstar_b) | ((k == vstar_b) & (f <= fstar_b))
        ps = plsc.cumsum(lax.convert_element_type(m, jnp.int32))
        idx = off + ps - 1
        plsc.store_scatter(buf_i, [idx], f, mask=m)
        plsc.store_scatter(buf_f, [idx], x, mask=m)
        return off + jnp.max(ps)
    off = lax.fori_loop(0, NV_B, fb_body, jnp.int32(0))
    stage_count(off)
    pltpu.sync_copy(buf_i, sh_buf_i.at[lb, j])
    pltpu.sync_copy(buf_f, sh_buf_f.at[lb, j])
    plsc.subcore_barrier()
    p1, p2, p3 = read_counts()
    pltpu.sync_copy(sh_buf_i.at[lb], stg_i4)
    pltpu.sync_copy(sh_buf_f.at[lb], stg_f4)

    def build_sel(i, _):
        p = i * 16 + iota
        pc = jnp.minimum(p, NUM_SELECT - 1)
        own = (lax.convert_element_type(pc >= p1, jnp.int32)
               + lax.convert_element_type(pc >= p2, jnp.int32)
               + lax.convert_element_type(pc >= p3, jnp.int32))
        pre = jnp.where(own == 0, 0,
                        jnp.where(own == 1, p1, jnp.where(own == 2, p2, p3)))
        plsc.store_scatter(flats_s, [p],
                           plsc.load_gather(stg_i4, [own, pc - pre]))
        plsc.store_scatter(vals_s, [p],
                           plsc.load_gather(stg_f4, [own, pc - pre]))
        return 0
    lax.fori_loop(0, 304 // 16, build_sel, 0)
    plsc.subcore_barrier()

    myv = []
    myf = []
    for v in range(5):
        e = jnp.minimum(j * EPW + v * 16 + iota, NUM_SELECT - 1)
        myv.append(plsc.load_gather(vals_s, [e]))
        myf.append(plsc.load_gather(flats_s, [e]))

    def rank_body(t, cnts):
        ts16 = jnp.full((16,), t, jnp.int32)
        vs = plsc.load_gather(vals_s, [ts16])
        fs = plsc.load_gather(flats_s, [ts16])
        out = []
        for v in range(5):
            win = (vs > myv[v]) | ((vs == myv[v]) & (fs < myf[v]))
            out.append(cnts[v] + lax.convert_element_type(win, jnp.int32))
        return tuple(out)
    cnts = lax.fori_loop(0, NUM_SELECT, rank_body,
                         tuple(zeros16 for _ in range(5)))

    for v in range(5):
        f = myf[v]
        p = jnp.clip(f // CP, 0, M_ROWS - 1)
        row = plsc.load_gather(cand_rows, [p])
        res_pos[pl.ds(v * 16, 16)] = cnts[v]
        res_score[pl.ds(v * 16, 16)] = myv[v]
        res_label[pl.ds(v * 16, 16)] = f % CP
        res_row[pl.ds(v * 16, 16)] = row
    pltpu.sync_copy(res_pos, sh_pos.at[lb, j])
    pltpu.sync_copy(res_score, sh_sc.at[lb, j])
    pltpu.sync_copy(res_label, sh_lab.at[lb, j])
    pltpu.sync_copy(res_row, sh_row.at[lb, j])
    plsc.subcore_barrier()

    @pl.when(j == 0)
    def _():
        pltpu.sync_copy(ts_hbm, ts_s)
        pltpu.sync_copy(boxes_hbm.at[pl.ds(b * (N * 4), N * 4)], bxall)
        pltpu.sync_copy(sh_pos.at[lb], stg_pos)
        pltpu.sync_copy(sh_sc.at[lb], stg_sc)
        pltpu.sync_copy(sh_lab.at[lb], stg_lab)
        pltpu.sync_copy(sh_row.at[lb], stg_row)

        def zero_out(i, _):
            plsc.store_scatter(srt_score, [i * 16 + iota],
                               jnp.zeros((16,), jnp.float32))
            plsc.store_scatter(srt_label, [i * 16 + iota], zeros16)
            plsc.store_scatter(srt_row, [jnp.minimum(i * 16 + iota, 319)],
                               zeros16)
            return 0
        lax.fori_loop(0, 384 // 16, zero_out, 0)

        def scat(i, _):
            e = i * 16 + iota
            w = e // RPW
            l = e % RPW
            valid = l < EPW
            pos = plsc.load_gather(stg_pos, [w, l])
            sc = plsc.load_gather(stg_sc, [w, l])
            lab = plsc.load_gather(stg_lab, [w, l])
            rw = plsc.load_gather(stg_row, [w, l])
            posc = jnp.clip(pos, 0, NUM_SELECT - 1)
            plsc.store_scatter(srt_score, [posc], sc, mask=valid)
            plsc.store_scatter(srt_label, [posc], lab, mask=valid)
            plsc.store_scatter(srt_row, [posc], rw, mask=valid)
            return 0
        lax.fori_loop(0, M_ROWS // 16, scat, 0)

        tsv = ts_s[pl.ds(0, 16)]
        hh = _extract(tsv, b * 2, jnp.int32(-2147483648))
        ww = _extract(tsv, b * 2 + 1, jnp.int32(-2147483648))
        wv = jnp.full((16,), lax.convert_element_type(ww, jnp.float32))
        hv = jnp.full((16,), lax.convert_element_type(hh, jnp.float32))

        def box_body(i, _):
            q = i * 16 + iota
            orow = q // 4
            col = q % 4
            row = plsc.load_gather(srt_row, [orow])
            v1 = plsc.load_gather(bxall, [row * 4 + col])
            v2 = plsc.load_gather(bxall, [row * 4 + (col ^ 2)])
            low = col < 2
            xy = jnp.where(low, v1 - 0.5 * v2, v2 + 0.5 * v1)
            scale = jnp.where((col & 1) == 0, wv, hv)
            plsc.store_scatter(bxo, [q], xy * scale)
            return 0
        lax.fori_loop(0, 1280 // 16, box_body, 0)

        pltpu.sync_copy(srt_score, scores_out.at[pl.ds(b * 384, 384)])
        pltpu.sync_copy(srt_label, labels_out.at[pl.ds(b * 384, 384)])
        pltpu.sync_copy(bxo, boxes_out.at[pl.ds(b * 1280, 1280)])


def _sc_select(prob_flat, rowmax, pred_boxes, target_sizes):
    mesh = plsc.VectorSubcoreMesh(core_axis_name="c", subcore_axis_name="s")
    f32 = jnp.float32
    i32 = jnp.int32
    kern = pl.kernel(
        _sc_body,
        mesh=mesh,
        compiler_params=pltpu.CompilerParams(needs_layout_passes=False),
        out_type=[
            jax.ShapeDtypeStruct((B * 384,), f32),
            jax.ShapeDtypeStruct((B * 384,), i32),
            jax.ShapeDtypeStruct((B * 1280,), f32),
        ],
        scratch_types=[
            pltpu.VMEM((NV_A * 16,), f32),
            pltpu.VMEM((4096,), i32),
            pltpu.VMEM((256,), i32),
            pltpu.VMEM((4, 256), i32),
            pltpu.VMEM((128,), i32),
            pltpu.VMEM((CAP,), i32),
            pltpu.VMEM((CAP,), f32),
            pltpu.VMEM((4, 128), i32),
            pltpu.VMEM((4, CAP), i32),
            pltpu.VMEM((4, CAP), f32),
            pltpu.VMEM((M_ROWS,), i32),
            pltpu.VMEM((RPW,), i32),
            pltpu.VMEM((RPW, CP), f32),
            pltpu.VMEM((304,), f32),
            pltpu.VMEM((304,), i32),
            pltpu.VMEM((128,), i32),
            pltpu.VMEM((128,), f32),
            pltpu.VMEM((128,), i32),
            pltpu.VMEM((128,), i32),
            pltpu.VMEM((4, 128), i32),
            pltpu.VMEM((4, 128), f32),
            pltpu.VMEM((4, 128), i32),
            pltpu.VMEM((4, 128), i32),
            pltpu.VMEM((384,), f32),
            pltpu.VMEM((384,), i32),
            pltpu.VMEM((320,), i32),
            pltpu.VMEM((N * 4,), f32),
            pltpu.VMEM((1280,), f32),
            pltpu.VMEM((B * 2,), i32),
            pltpu.SemaphoreType.DMA,
            pltpu.VMEM_SHARED((4, 4, 256), i32),
            pltpu.VMEM_SHARED((4, 4, 128), i32),
            pltpu.VMEM_SHARED((4, 4, CAP), i32),
            pltpu.VMEM_SHARED((4, 4, CAP), f32),
            pltpu.VMEM_SHARED((4, 4, 128), i32),
            pltpu.VMEM_SHARED((4, 4, 128), f32),
            pltpu.VMEM_SHARED((4, 4, 128), i32),
            pltpu.VMEM_SHARED((4, 4, 128), i32),
        ],
    )
    return kern(prob_flat, rowmax, pred_boxes, target_sizes)


@jax.jit
def kernel(pred_logits, pred_boxes, target_sizes, pos_map):
    prob, rowmax = _compute_prob(pred_logits, pos_map)
    rowmax_pad = jnp.concatenate(
        [rowmax, jnp.full((B, NW * ROWS_W - N), -jnp.inf, jnp.float32)],
        axis=1).reshape(-1)
    scores_p, labels_p, boxes_p = _sc_select(
        prob.reshape(B * N, CP), rowmax_pad, pred_boxes.reshape(-1),
        target_sizes.reshape(-1))
    return (scores_p.reshape(B, 384)[:, :NUM_SELECT],
            labels_p.reshape(B, 384)[:, :NUM_SELECT],
            boxes_p.reshape(B, 320, 4)[:, :NUM_SELECT, :])
